# baseline (speedup 1.0000x reference)
import functools

import jax
import jax.numpy as jnp
from jax.experimental import pallas as pl
from jax.experimental.pallas import tpu as pltpu

MD = 7
PATCH = 32
NUM_MA = 24


def _diff_kernel(x_ref, out_ref, ps_ref):
    h = pl.program_id(1)
    nh = pl.num_programs(1)
    C, T = x_ref.shape[1], x_ref.shape[2]

    part = None
    for c in range(C):
        prev = x_ref[0, c, 0]
        for t in range(1, T):
            cur = x_ref[0, c, t]
            d = jnp.abs(cur - prev)
            part = d if part is None else part + d
            prev = cur

    row = jnp.concatenate(
        [part[:, j * PATCH:(j + 1) * PATCH].sum(axis=1, keepdims=True)
         for j in range(MD)], axis=1).sum(axis=0, keepdims=True)
    ps_ref[pl.ds(h, 1), :] = row

    @pl.when(h == nh - 1)
    def _():
        ps = ps_ref[...]
        idx = jax.lax.broadcasted_iota(jnp.int32, (MD, MD), 0) * MD + \
              jax.lax.broadcasted_iota(jnp.int32, (MD, MD), 1)
        a = ps[:, :, None, None]
        b = ps[None, None, :, :]
        ia = idx[:, :, None, None]
        ib = idx[None, None, :, :]
        beats = (b > a) | ((b == a) & (ib < ia))
        rank = beats.astype(jnp.int32).sum(axis=(2, 3))
        out_ref[0] = (rank < NUM_MA).astype(jnp.float32)


@jax.jit
def kernel(x):
    B, C, T, H, W = x.shape
    return pl.pallas_call(
        _diff_kernel,
        grid=(B, H // PATCH),
        in_specs=[pl.BlockSpec((1, C, T, PATCH, W),
                               lambda b, h: (b, 0, 0, h, 0))],
        out_specs=pl.BlockSpec((1, MD, MD), lambda b, h: (b, 0, 0)),
        out_shape=jax.ShapeDtypeStruct((B, MD, MD), jnp.float32),
        scratch_shapes=[pltpu.VMEM((MD, MD), jnp.float32)],
    )(x)
